# baseline (speedup 1.0000x reference)
import jax
import jax.numpy as jnp
from jax.experimental import pallas as pl

OUT = 4096
IN = 4096
BLOCK_ROWS = 256


def _soft_threshold_kernel(w_ref, t_ref, o_ref):
    w = w_ref[...]
    s = jax.nn.sigmoid(t_ref[...])
    o_ref[...] = jnp.sign(w) * jnp.maximum(jnp.abs(w) - s, 0.0)


def kernel(weight, threshold, alpha):
    del alpha
    grid = (OUT // BLOCK_ROWS,)
    out = pl.pallas_call(
        _soft_threshold_kernel,
        grid=grid,
        in_specs=[
            pl.BlockSpec((BLOCK_ROWS, IN), lambda i: (i, 0)),
            pl.BlockSpec((BLOCK_ROWS, 1), lambda i: (i, 0)),
        ],
        out_specs=pl.BlockSpec((BLOCK_ROWS, IN), lambda i: (i, 0)),
        out_shape=jax.ShapeDtypeStruct((OUT, IN), jnp.float32),
    )(weight, threshold)
    return out


# speedup vs baseline: 1.1963x; 1.1963x over previous
import jax
import jax.numpy as jnp
from jax.experimental import pallas as pl

OUT = 4096
IN = 4096
BLOCK_ROWS = 256


def _soft_threshold_kernel(w_ref, t_ref, o_ref):
    w = w_ref[...]
    s = jax.nn.sigmoid(t_ref[...])
    bits = jax.lax.bitcast_convert_type(w, jnp.uint32)
    mag = jax.lax.bitcast_convert_type(
        bits & jnp.uint32(0x7FFFFFFF), jnp.float32
    )
    sgn = bits & jnp.uint32(0x80000000)
    m = jnp.maximum(mag - s, 0.0)
    o_ref[...] = jax.lax.bitcast_convert_type(
        jax.lax.bitcast_convert_type(m, jnp.uint32) | sgn, jnp.float32
    )


def kernel(weight, threshold, alpha):
    del alpha
    grid = (OUT // BLOCK_ROWS,)
    out = pl.pallas_call(
        _soft_threshold_kernel,
        grid=grid,
        in_specs=[
            pl.BlockSpec((BLOCK_ROWS, IN), lambda i: (i, 0)),
            pl.BlockSpec((BLOCK_ROWS, 1), lambda i: (i, 0)),
        ],
        out_specs=pl.BlockSpec((BLOCK_ROWS, IN), lambda i: (i, 0)),
        out_shape=jax.ShapeDtypeStruct((OUT, IN), jnp.float32),
    )(weight, threshold)
    return out


# speedup vs baseline: 1.2386x; 1.0354x over previous
import jax
import jax.numpy as jnp
from jax.experimental import pallas as pl
from jax.experimental.pallas import tpu as pltpu

OUT = 4096
IN = 4096
BLOCK_ROWS = 256


CHUNK_ROWS = 8


def _soft_threshold_kernel(w_ref, t_ref, o_ref):
    s_all = jax.nn.sigmoid(t_ref[...])
    for c in range(BLOCK_ROWS // CHUNK_ROWS):
        r0 = c * CHUNK_ROWS
        w = w_ref[r0:r0 + CHUNK_ROWS, :]
        s = s_all[r0:r0 + CHUNK_ROWS, :]
        bits = jax.lax.bitcast_convert_type(w, jnp.uint32)
        mag = jax.lax.bitcast_convert_type(
            bits & jnp.uint32(0x7FFFFFFF), jnp.float32
        )
        sgn = bits & jnp.uint32(0x80000000)
        m = jnp.maximum(mag - s, 0.0)
        o_ref[r0:r0 + CHUNK_ROWS, :] = jax.lax.bitcast_convert_type(
            jax.lax.bitcast_convert_type(m, jnp.uint32) | sgn, jnp.float32
        )


def kernel(weight, threshold, alpha):
    del alpha
    grid = (OUT // BLOCK_ROWS,)
    out = pl.pallas_call(
        _soft_threshold_kernel,
        grid=grid,
        in_specs=[
            pl.BlockSpec((BLOCK_ROWS, IN), lambda i: (i, 0)),
            pl.BlockSpec((BLOCK_ROWS, 1), lambda i: (i, 0)),
        ],
        out_specs=pl.BlockSpec((BLOCK_ROWS, IN), lambda i: (i, 0)),
        out_shape=jax.ShapeDtypeStruct((OUT, IN), jnp.float32),
        compiler_params=pltpu.CompilerParams(
            dimension_semantics=("parallel",),
        ),
    )(weight, threshold)
    return out


# speedup vs baseline: 1.2637x; 1.0203x over previous
import jax
import jax.numpy as jnp
from jax.experimental import pallas as pl
from jax.experimental.pallas import tpu as pltpu

OUT = 4096
IN = 4096
BLOCK_ROWS = 512


CHUNK_ROWS = 8


def _soft_threshold_kernel(w_ref, t_ref, o_ref):
    s_all = jax.nn.sigmoid(t_ref[...])
    for c in range(BLOCK_ROWS // CHUNK_ROWS):
        r0 = c * CHUNK_ROWS
        w = w_ref[r0:r0 + CHUNK_ROWS, :]
        s = s_all[r0:r0 + CHUNK_ROWS, :]
        bits = jax.lax.bitcast_convert_type(w, jnp.uint32)
        mag = jax.lax.bitcast_convert_type(
            bits & jnp.uint32(0x7FFFFFFF), jnp.float32
        )
        sgn = bits & jnp.uint32(0x80000000)
        m = jnp.maximum(mag - s, 0.0)
        o_ref[r0:r0 + CHUNK_ROWS, :] = jax.lax.bitcast_convert_type(
            jax.lax.bitcast_convert_type(m, jnp.uint32) | sgn, jnp.float32
        )


def kernel(weight, threshold, alpha):
    del alpha
    grid = (OUT // BLOCK_ROWS,)
    out = pl.pallas_call(
        _soft_threshold_kernel,
        grid=grid,
        in_specs=[
            pl.BlockSpec((BLOCK_ROWS, IN), lambda i: (i, 0)),
            pl.BlockSpec((BLOCK_ROWS, 1), lambda i: (i, 0)),
        ],
        out_specs=pl.BlockSpec((BLOCK_ROWS, IN), lambda i: (i, 0)),
        out_shape=jax.ShapeDtypeStruct((OUT, IN), jnp.float32),
        compiler_params=pltpu.CompilerParams(
            dimension_semantics=("parallel",),
        ),
    )(weight, threshold)
    return out
